# baseline (speedup 1.0000x reference)
import jax
import jax.numpy as jnp
from jax import lax
from jax.experimental import pallas as pl
from jax.experimental.pallas import tpu as pltpu
from jax.experimental.pallas import tpu_sc as plsc

NC = 2
NS = 16
NW = NC * NS
F32 = jnp.float32


def _ceil_to(v, m):
    return -(-v // m) * m


def kernel(tokens, edge_index, emb_table, W, b):
    N = tokens.shape[0]
    E = edge_index.shape[1]
    V, D = emb_table.shape

    TCH = 80
    NPAD = _ceil_to(N, NW * TCH)
    TPT = NPAD // NW
    TKC = TPT // TCH

    ECH = 128
    EPAD = _ceil_to(E, 2 * NW * ECH)
    EPT = EPAD // NW
    EKC = EPT // ECH

    RPS = NPAD // NS

    tokens_p = jnp.concatenate(
        [tokens.astype(jnp.int32), jnp.zeros((NPAD - N,), jnp.int32)])
    src_r = jnp.concatenate(
        [edge_index[0].astype(jnp.int32), jnp.zeros((EPAD - E,), jnp.int32)]
    ).reshape(NW, EKC, ECH)
    dst_r = jnp.concatenate(
        [edge_index[1].astype(jnp.int32), jnp.full((EPAD - E,), N, jnp.int32)]
    ).reshape(NW, EKC, ECH)

    mesh = plsc.VectorSubcoreMesh(
        core_axis_name="c", subcore_axis_name="s",
        num_cores=NC, num_subcores=NS)

    def sc_gather_deg(tokens_hbm, dstr_hbm, emb_hbm, x_out, deg_out,
                      tok_all, dst_all, rows0, rows1, ones, zbuf, deg_sh,
                      sem_g0, sem_g1, sem_h):
        c = lax.axis_index("c")
        s = lax.axis_index("s")
        wid = s * NC + c

        def fz(i, _):
            zbuf[pl.ds(i * 16, 16)] = jnp.zeros((16,), F32)
            return 0
        lax.fori_loop(0, RPS // 16, fz, 0)

        def fo(i, _):
            ones[pl.ds(i * 16, 16)] = jnp.ones((16,), F32)
            return 0
        lax.fori_loop(0, ECH // 16, fo, 0)

        pltpu.sync_copy(zbuf, deg_sh.at[pl.ds(s * RPS, RPS)])

        pltpu.sync_copy(tokens_hbm.at[pl.ds(wid * TPT, TPT)], tok_all)
        pltpu.sync_copy(dstr_hbm.at[wid], dst_all)
        plsc.subcore_barrier()

        def hchunk(k, _):
            pltpu.async_copy(ones, deg_sh.at[dst_all.at[k]], sem_h, add=True)
            return 0
        lax.fori_loop(0, EKC, hchunk, 0)

        bufs = (rows0, rows1)
        sems = (sem_g0, sem_g1)
        pltpu.async_copy(emb_hbm.at[tok_all.at[pl.ds(0, TCH)]], rows0, sem_g0)
        pltpu.async_copy(emb_hbm.at[tok_all.at[pl.ds(TCH, TCH)]], rows1,
                         sem_g1)
        for k in range(TKC):
            buf, sem = bufs[k % 2], sems[k % 2]
            pltpu.make_async_copy(emb_hbm.at[pl.ds(0, TCH)], buf, sem).wait()
            pltpu.sync_copy(buf, x_out.at[pl.ds(wid * TPT + k * TCH, TCH)])
            if k + 2 < TKC:
                idx = tok_all.at[pl.ds((k + 2) * TCH, TCH)]
                pltpu.async_copy(emb_hbm.at[idx], buf, sem)

        pltpu.make_async_copy(dstr_hbm.at[wid], dst_all, sem_h).wait()
        plsc.subcore_barrier()
        pltpu.sync_copy(deg_sh.at[pl.ds(s * RPS, RPS)],
                        deg_out.at[c, pl.ds(s * RPS, RPS)])

    sc_a = pl.kernel(
        sc_gather_deg,
        out_type=(jax.ShapeDtypeStruct((NPAD, D), F32),
                  jax.ShapeDtypeStruct((NC, NPAD), F32)),
        mesh=mesh,
        scratch_types=[
            pltpu.VMEM((TPT,), jnp.int32),
            pltpu.VMEM((EKC, ECH), jnp.int32),
            pltpu.VMEM((TCH, D), F32),
            pltpu.VMEM((TCH, D), F32),
            pltpu.VMEM((ECH,), F32),
            pltpu.VMEM((RPS,), F32),
            pltpu.VMEM_SHARED((NPAD,), F32),
            pltpu.SemaphoreType.DMA,
            pltpu.SemaphoreType.DMA,
            pltpu.SemaphoreType.DMA,
        ],
    )
    x_pad, deg_parts = sc_a(tokens_p, dst_r, emb_table)
    deg_t = jnp.transpose(deg_parts)

    BLK = 1024
    G = NPAD // BLK

    def tc_matmul_scale(x_ref, w_ref, dp_ref, y_ref):
        dp = dp_ref[...]
        dinv = lax.rsqrt(dp[:, 0:1] + dp[:, 1:2] + 1.0)
        xw = jnp.dot(x_ref[...], w_ref[...], preferred_element_type=F32)
        y_ref[...] = xw * dinv

    y_pad = pl.pallas_call(
        tc_matmul_scale,
        grid=(G,),
        in_specs=[pl.BlockSpec((BLK, D), lambda g: (g, 0)),
                  pl.BlockSpec((D, D), lambda g: (0, 0)),
                  pl.BlockSpec((BLK, NC), lambda g: (g, 0))],
        out_specs=pl.BlockSpec((BLK, D), lambda g: (g, 0)),
        out_shape=jax.ShapeDtypeStruct((NPAD, D), F32),
    )(x_pad, W, deg_t)

    def sc_edge_agg(y_hbm, srcr_hbm, dst_hbm, acc_out,
                    src_all, didx0, didx1, rows0, rows1, zrow, acc_sh,
                    sem_g0, sem_g1, sem_d0, sem_d1):
        c = lax.axis_index("c")
        s = lax.axis_index("s")
        wid = s * NC + c

        def fz(i, _):
            zrow[i // 8, pl.ds((i % 8) * 16, 16)] = jnp.zeros((16,), F32)
            return 0
        lax.fori_loop(0, 8 * (D // 16), fz, 0)

        def zc(i, _):
            pltpu.sync_copy(zrow, acc_sh.at[pl.ds(s * RPS + i * 8, 8)])
            return 0
        lax.fori_loop(0, RPS // 8, zc, 0)

        pltpu.sync_copy(srcr_hbm.at[wid], src_all)
        plsc.subcore_barrier()

        SPL = 4
        SC_ = ECH // SPL

        def stage(k, didx, sem_d, buf, sem_g):
            base = wid * EPT + k * ECH
            pltpu.async_copy(dst_hbm.at[pl.ds(base, ECH)], didx, sem_d)
            for p in range(SPL):
                pltpu.async_copy(
                    y_hbm.at[src_all.at[k, pl.ds(p * SC_, SC_)]],
                    buf.at[pl.ds(p * SC_, SC_)], sem_g)

        stage(0, didx0, sem_d0, rows0, sem_g0)
        stage(1, didx1, sem_d1, rows1, sem_g1)

        def slot(k, didx, sem_d, buf, sem_g):
            pltpu.make_async_copy(y_hbm.at[pl.ds(0, ECH)], buf, sem_g).wait()
            pltpu.make_async_copy(dst_hbm.at[pl.ds(0, ECH)], didx,
                                  sem_d).wait()
            pltpu.sync_copy(buf, acc_sh.at[didx], add=True)

            @pl.when(k + 2 < EKC)
            def _():
                stage(k + 2, didx, sem_d, buf, sem_g)

        def echunk(i, _):
            slot(2 * i, didx0, sem_d0, rows0, sem_g0)
            slot(2 * i + 1, didx1, sem_d1, rows1, sem_g1)
            return 0
        lax.fori_loop(0, EKC // 2, echunk, 0)

        plsc.subcore_barrier()

        def oc(i, _):
            r0 = s * RPS + i * ECH
            pltpu.sync_copy(acc_sh.at[pl.ds(r0, ECH)],
                            acc_out.at[c, pl.ds(r0, ECH)])
            return 0
        lax.fori_loop(0, RPS // ECH, oc, 0)

    sc_c = pl.kernel(
        sc_edge_agg,
        out_type=jax.ShapeDtypeStruct((NC, NPAD, D), F32),
        mesh=mesh,
        scratch_types=[
            pltpu.VMEM((EKC, ECH), jnp.int32),
            pltpu.VMEM((ECH,), jnp.int32),
            pltpu.VMEM((ECH,), jnp.int32),
            pltpu.VMEM((ECH, D), F32),
            pltpu.VMEM((ECH, D), F32),
            pltpu.VMEM((8, D), F32),
            pltpu.VMEM_SHARED((NPAD, D), F32),
            pltpu.SemaphoreType.DMA,
            pltpu.SemaphoreType.DMA,
            pltpu.SemaphoreType.DMA,
            pltpu.SemaphoreType.DMA,
        ],
    )
    acc_parts = sc_c(y_pad, src_r, dst_r.reshape(EPAD))

    OBLK = 1000
    OG = N // OBLK

    def tc_combine(a0_ref, a1_ref, y_ref, dp_ref, b_ref, o_ref):
        dp = dp_ref[...]
        dinv = lax.rsqrt(dp[:, 0:1] + dp[:, 1:2] + 1.0)
        acc = a0_ref[...] + a1_ref[...] + y_ref[...]
        o_ref[...] = jnp.maximum(acc * dinv + b_ref[...], 0.0)

    out = pl.pallas_call(
        tc_combine,
        grid=(OG,),
        in_specs=[pl.BlockSpec((OBLK, D), lambda g: (g, 0)),
                  pl.BlockSpec((OBLK, D), lambda g: (g, 0)),
                  pl.BlockSpec((OBLK, D), lambda g: (g, 0)),
                  pl.BlockSpec((OBLK, NC), lambda g: (g, 0)),
                  pl.BlockSpec((1, D), lambda g: (0, 0))],
        out_specs=pl.BlockSpec((OBLK, D), lambda g: (g, 0)),
        out_shape=jax.ShapeDtypeStruct((N, D), F32),
    )(acc_parts[0], acc_parts[1], y_pad, deg_t, b.reshape(1, D))

    return out


# speedup vs baseline: 1.0075x; 1.0075x over previous
import jax
import jax.numpy as jnp
from jax import lax
from jax.experimental import pallas as pl
from jax.experimental.pallas import tpu as pltpu
from jax.experimental.pallas import tpu_sc as plsc

NC = 2
NS = 16
NW = NC * NS
F32 = jnp.float32


def _ceil_to(v, m):
    return -(-v // m) * m


def kernel(tokens, edge_index, emb_table, W, b):
    N = tokens.shape[0]
    E = edge_index.shape[1]
    V, D = emb_table.shape

    TCH = 80
    NPAD = _ceil_to(N, NW * TCH)
    TPT = NPAD // NW
    TKC = TPT // TCH

    ECH = 128
    EPAD = _ceil_to(E, 2 * NW * ECH)
    EPT = EPAD // NW
    EKC = EPT // ECH

    RPS = NPAD // NS

    tokens_p = jnp.concatenate(
        [tokens.astype(jnp.int32), jnp.zeros((NPAD - N,), jnp.int32)])
    src_r = jnp.concatenate(
        [edge_index[0].astype(jnp.int32), jnp.zeros((EPAD - E,), jnp.int32)]
    ).reshape(NW, EKC, ECH)
    dst_r = jnp.concatenate(
        [edge_index[1].astype(jnp.int32), jnp.full((EPAD - E,), N, jnp.int32)]
    ).reshape(NW, EKC, ECH)

    mesh = plsc.VectorSubcoreMesh(
        core_axis_name="c", subcore_axis_name="s",
        num_cores=NC, num_subcores=NS)

    def sc_gather_deg(tokens_hbm, dstr_hbm, emb_hbm, x_out, deg_out,
                      tok_all, dst_all, rows0, rows1, ones, zbuf, deg_sh,
                      sem_g0, sem_g1, sem_h):
        c = lax.axis_index("c")
        s = lax.axis_index("s")
        wid = s * NC + c

        def fz(i, _):
            zbuf[pl.ds(i * 16, 16)] = jnp.zeros((16,), F32)
            return 0
        lax.fori_loop(0, RPS // 16, fz, 0)

        def fo(i, _):
            ones[pl.ds(i * 16, 16)] = jnp.ones((16,), F32)
            return 0
        lax.fori_loop(0, ECH // 16, fo, 0)

        pltpu.sync_copy(zbuf, deg_sh.at[pl.ds(s * RPS, RPS)])

        pltpu.sync_copy(tokens_hbm.at[pl.ds(wid * TPT, TPT)], tok_all)
        pltpu.sync_copy(dstr_hbm.at[wid], dst_all)
        plsc.subcore_barrier()

        def hchunk(k, _):
            pltpu.async_copy(ones, deg_sh.at[dst_all.at[k]], sem_h, add=True)
            return 0
        lax.fori_loop(0, EKC, hchunk, 0)

        bufs = (rows0, rows1)
        sems = (sem_g0, sem_g1)
        pltpu.async_copy(emb_hbm.at[tok_all.at[pl.ds(0, TCH)]], rows0, sem_g0)
        pltpu.async_copy(emb_hbm.at[tok_all.at[pl.ds(TCH, TCH)]], rows1,
                         sem_g1)
        for k in range(TKC):
            buf, sem = bufs[k % 2], sems[k % 2]
            pltpu.make_async_copy(emb_hbm.at[pl.ds(0, TCH)], buf, sem).wait()
            pltpu.sync_copy(buf, x_out.at[pl.ds(wid * TPT + k * TCH, TCH)])
            if k + 2 < TKC:
                idx = tok_all.at[pl.ds((k + 2) * TCH, TCH)]
                pltpu.async_copy(emb_hbm.at[idx], buf, sem)

        pltpu.make_async_copy(dstr_hbm.at[wid], dst_all, sem_h).wait()
        plsc.subcore_barrier()
        pltpu.sync_copy(deg_sh.at[pl.ds(s * RPS, RPS)],
                        deg_out.at[c, pl.ds(s * RPS, RPS)])

    sc_a = pl.kernel(
        sc_gather_deg,
        out_type=(jax.ShapeDtypeStruct((NPAD, D), F32),
                  jax.ShapeDtypeStruct((NC, NPAD), F32)),
        mesh=mesh,
        scratch_types=[
            pltpu.VMEM((TPT,), jnp.int32),
            pltpu.VMEM((EKC, ECH), jnp.int32),
            pltpu.VMEM((TCH, D), F32),
            pltpu.VMEM((TCH, D), F32),
            pltpu.VMEM((ECH,), F32),
            pltpu.VMEM((RPS,), F32),
            pltpu.VMEM_SHARED((NPAD,), F32),
            pltpu.SemaphoreType.DMA,
            pltpu.SemaphoreType.DMA,
            pltpu.SemaphoreType.DMA,
        ],
    )
    x_pad, deg_parts = sc_a(tokens_p, dst_r, emb_table)
    deg_t = jnp.transpose(deg_parts)

    BLK = 1024
    G = NPAD // BLK

    def tc_matmul_scale(x_ref, w_ref, dp_ref, y_ref):
        dp = dp_ref[...]
        dinv = lax.rsqrt(dp[:, 0:1] + dp[:, 1:2] + 1.0)
        xw = jnp.dot(x_ref[...], w_ref[...], preferred_element_type=F32)
        y_ref[...] = xw * dinv

    y_pad = pl.pallas_call(
        tc_matmul_scale,
        grid=(G,),
        in_specs=[pl.BlockSpec((BLK, D), lambda g: (g, 0)),
                  pl.BlockSpec((D, D), lambda g: (0, 0)),
                  pl.BlockSpec((BLK, NC), lambda g: (g, 0))],
        out_specs=pl.BlockSpec((BLK, D), lambda g: (g, 0)),
        out_shape=jax.ShapeDtypeStruct((NPAD, D), F32),
    )(x_pad, W, deg_t)

    def sc_edge_agg(y_hbm, srcr_hbm, dst_hbm, acc_out,
                    src_all, didx0, didx1, rows0, rows1, zrow, acc_sh,
                    sem_g0, sem_g1, sem_d0, sem_d1):
        c = lax.axis_index("c")
        s = lax.axis_index("s")
        wid = s * NC + c

        def fz(i, _):
            zrow[i // 8, pl.ds((i % 8) * 16, 16)] = jnp.zeros((16,), F32)
            return 0
        lax.fori_loop(0, 32 * (D // 16), fz, 0)

        def zc(i, _):
            pltpu.async_copy(zrow, acc_sh.at[pl.ds(s * RPS + i * 32, 32)],
                             sem_d0)
            return 0
        lax.fori_loop(0, RPS // 32, zc, 0)

        pltpu.sync_copy(srcr_hbm.at[wid], src_all)

        def zw(i, _):
            pltpu.make_async_copy(
                zrow, acc_sh.at[pl.ds(s * RPS + i * 32, 32)], sem_d0).wait()
            return 0
        lax.fori_loop(0, RPS // 32, zw, 0)
        plsc.subcore_barrier()

        SPL = 4
        SC_ = ECH // SPL

        def stage(k, didx, sem_d, buf, sem_g):
            base = wid * EPT + k * ECH
            pltpu.async_copy(dst_hbm.at[pl.ds(base, ECH)], didx, sem_d)
            for p in range(SPL):
                pltpu.async_copy(
                    y_hbm.at[src_all.at[k, pl.ds(p * SC_, SC_)]],
                    buf.at[pl.ds(p * SC_, SC_)], sem_g)

        stage(0, didx0, sem_d0, rows0, sem_g0)
        stage(1, didx1, sem_d1, rows1, sem_g1)

        def slot(k, didx, sem_d, buf, sem_g):
            pltpu.make_async_copy(y_hbm.at[pl.ds(0, ECH)], buf, sem_g).wait()
            pltpu.make_async_copy(dst_hbm.at[pl.ds(0, ECH)], didx,
                                  sem_d).wait()
            pltpu.sync_copy(buf, acc_sh.at[didx], add=True)

            @pl.when(k + 2 < EKC)
            def _():
                stage(k + 2, didx, sem_d, buf, sem_g)

        def echunk(i, _):
            slot(2 * i, didx0, sem_d0, rows0, sem_g0)
            slot(2 * i + 1, didx1, sem_d1, rows1, sem_g1)
            return 0
        lax.fori_loop(0, EKC // 2, echunk, 0)

        plsc.subcore_barrier()

        def oc(i, _):
            r0 = s * RPS + i * 128
            pltpu.async_copy(acc_sh.at[pl.ds(r0, 128)],
                             acc_out.at[c, pl.ds(r0, 128)], sem_d0)
            return 0
        lax.fori_loop(0, RPS // 128, oc, 0)

        def ow(i, _):
            r0 = s * RPS + i * 128
            pltpu.make_async_copy(acc_sh.at[pl.ds(r0, 128)],
                                  acc_out.at[c, pl.ds(r0, 128)],
                                  sem_d0).wait()
            return 0
        lax.fori_loop(0, RPS // 128, ow, 0)

    sc_c = pl.kernel(
        sc_edge_agg,
        out_type=jax.ShapeDtypeStruct((NC, NPAD, D), F32),
        mesh=mesh,
        scratch_types=[
            pltpu.VMEM((EKC, ECH), jnp.int32),
            pltpu.VMEM((ECH,), jnp.int32),
            pltpu.VMEM((ECH,), jnp.int32),
            pltpu.VMEM((ECH, D), F32),
            pltpu.VMEM((ECH, D), F32),
            pltpu.VMEM((32, D), F32),
            pltpu.VMEM_SHARED((NPAD, D), F32),
            pltpu.SemaphoreType.DMA,
            pltpu.SemaphoreType.DMA,
            pltpu.SemaphoreType.DMA,
            pltpu.SemaphoreType.DMA,
        ],
    )
    acc_parts = sc_c(y_pad, src_r, dst_r.reshape(EPAD))

    OBLK = 1000
    OG = N // OBLK

    def tc_combine(a0_ref, a1_ref, y_ref, dp_ref, b_ref, o_ref):
        dp = dp_ref[...]
        dinv = lax.rsqrt(dp[:, 0:1] + dp[:, 1:2] + 1.0)
        acc = a0_ref[...] + a1_ref[...] + y_ref[...]
        o_ref[...] = jnp.maximum(acc * dinv + b_ref[...], 0.0)

    out = pl.pallas_call(
        tc_combine,
        grid=(OG,),
        in_specs=[pl.BlockSpec((OBLK, D), lambda g: (g, 0)),
                  pl.BlockSpec((OBLK, D), lambda g: (g, 0)),
                  pl.BlockSpec((OBLK, D), lambda g: (g, 0)),
                  pl.BlockSpec((OBLK, NC), lambda g: (g, 0)),
                  pl.BlockSpec((1, D), lambda g: (0, 0))],
        out_specs=pl.BlockSpec((OBLK, D), lambda g: (g, 0)),
        out_shape=jax.ShapeDtypeStruct((N, D), F32),
    )(acc_parts[0], acc_parts[1], y_pad, deg_t, b.reshape(1, D))

    return out
